# baseline (speedup 1.0000x reference)
import functools

import jax
import jax.numpy as jnp
from jax import lax
from jax.experimental import pallas as pl
from jax.experimental.pallas import tpu as pltpu
from jax.experimental.pallas import tpu_sc as plsc

_T, _H, _E, _K = 2048, 768, 64, 2
_TK = _T * _K
_RB = 128
_NT = _TK // _RB + _E
_PR = _NT * _RB
_NTP = 128
_NW = 32
_CW = _T // _NW


def _router_body(x_ref, wr_ref, x0_ref, x1_ref, pos_ref, meta_ref):
    x = x_ref[...]
    logits = jnp.dot(x, wr_ref[...], preferred_element_type=jnp.float32)
    lane = lax.broadcasted_iota(jnp.int32, (_T, _E), 1)
    m0 = jnp.max(logits, axis=1, keepdims=True)
    i0 = jnp.min(jnp.where(logits == m0, lane, _E), axis=1, keepdims=True)
    oh0 = lane == i0
    masked = jnp.where(oh0, -jnp.inf, logits)
    m1 = jnp.max(masked, axis=1, keepdims=True)
    i1 = jnp.min(jnp.where(masked == m1, lane, _E), axis=1, keepdims=True)
    oh1 = lane == i1
    w0 = jax.nn.sigmoid(m0 - m1)
    w1 = 1.0 - w0
    x0_ref[...] = (x * w0).astype(jnp.bfloat16)
    x1_ref[...] = (x * w1).astype(jnp.bfloat16)

    a = oh0.astype(jnp.float32) + oh1.astype(jnp.float32)
    rows = lax.broadcasted_iota(jnp.int32, (_T, _T), 0)
    cols = lax.broadcasted_iota(jnp.int32, (_T, _T), 1)
    lstrict = (cols < rows).astype(jnp.float32)
    cum = jnp.dot(lstrict, a, preferred_element_type=jnp.float32)
    counts = jnp.sum(a, axis=0)
    counts_i = counts.astype(jnp.int32)
    nb = (counts_i + (_RB - 1)) // _RB
    ef = lax.broadcasted_iota(jnp.int32, (_E, _E), 0)
    ee = lax.broadcasted_iota(jnp.int32, (_E, _E), 1)
    mstrict = (ef < ee).astype(jnp.float32)
    baseblk = jnp.dot(nb.astype(jnp.float32)[None, :], mstrict,
                      preferred_element_type=jnp.float32)[0]
    poffs = baseblk * float(_RB)
    val = poffs[None, :] + cum
    pos0 = jnp.sum(jnp.where(oh0, val, 0.0), axis=1)
    pos1 = jnp.sum(jnp.where(oh1, val, 0.0), axis=1)
    pos_ref[...] = jnp.concatenate(
        [pos0[None, :], pos1[None, :]], axis=0).astype(jnp.int32)

    baseblk_i = baseblk.astype(jnp.int32)
    total = jnp.sum(nb)
    tau = lax.broadcasted_iota(jnp.int32, (_NTP, _E), 0)
    eidx = lax.broadcasted_iota(jnp.int32, (_NTP, _E), 1)
    cand = (nb[None, :] > 0) & (baseblk_i[None, :] <= tau)
    et = jnp.max(jnp.where(cand, eidx, 0), axis=1)
    sel = (eidx == et[:, None]).astype(jnp.int32)
    pe = jnp.sum(sel * (baseblk_i * _RB)[None, :], axis=1)
    ce = jnp.sum(sel * counts_i[None, :], axis=1)
    tvec = lax.broadcasted_iota(jnp.int32, (_NTP,), 0)
    valid = tvec < total
    rb = jnp.where(valid, tvec, total - 1)
    lo = jnp.where(valid, tvec * _RB, 0)
    hi = jnp.where(valid, jnp.minimum(pe + ce, (tvec + 1) * _RB), 0)
    zero = jnp.zeros((4, _NTP), jnp.int32)
    meta_ref[...] = jnp.concatenate(
        [rb[None, :], et[None, :], lo[None, :], hi[None, :], zero], axis=0)


def _run_router(x, w_router):
    return pl.pallas_call(
        _router_body,
        out_shape=(
            jax.ShapeDtypeStruct((_T, _H), jnp.bfloat16),
            jax.ShapeDtypeStruct((_T, _H), jnp.bfloat16),
            jax.ShapeDtypeStruct((_K, _T), jnp.int32),
            jax.ShapeDtypeStruct((8, _NTP), jnp.int32),
        ),
    )(x, w_router)


@functools.cache
def _sc_mesh():
    return plsc.VectorSubcoreMesh(core_axis_name="c", subcore_axis_name="s")


@functools.cache
def _make_sc_scatter():
    @functools.partial(
        pl.kernel,
        mesh=_sc_mesh(),
        out_type=jax.ShapeDtypeStruct((_PR, _H // 2), jnp.int32),
        scratch_types=[
            pltpu.VMEM((_CW,), jnp.int32),
            pltpu.VMEM((_CW,), jnp.int32),
            pltpu.VMEM((_CW, _H // 2), jnp.int32),
            pltpu.VMEM((_CW, _H // 2), jnp.int32),
            pltpu.SemaphoreType.DMA,
            pltpu.SemaphoreType.DMA,
        ],
    )
    def _sc_scatter(x0_hbm, x1_hbm, pos_hbm, xs_hbm,
                    idx0, idx1, buf0, buf1, sem0, sem1):
        wid = lax.axis_index("s") * 2 + lax.axis_index("c")
        base = wid * _CW
        pltpu.sync_copy(pos_hbm.at[0, pl.ds(base, _CW)], idx0)
        pltpu.sync_copy(pos_hbm.at[1, pl.ds(base, _CW)], idx1)
        pltpu.sync_copy(x0_hbm.at[pl.ds(base, _CW), :], buf0)
        pltpu.sync_copy(x1_hbm.at[pl.ds(base, _CW), :], buf1)
        c0 = pltpu.async_copy(buf0, xs_hbm.at[idx0], sem0)
        c1 = pltpu.async_copy(buf1, xs_hbm.at[idx1], sem1)
        c0.wait()
        c1.wait()

    return _sc_scatter


def _gemm_body(rb_ref, et_ref, lo_ref, hi_ref, xs_ref, we_ref, out_ref):
    t = pl.program_id(0)
    rb = rb_ref[t]
    lo = lo_ref[t]
    hi = hi_ref[t]

    @pl.when(lo < hi)
    def _compute():
        row = rb * _RB + lax.broadcasted_iota(jnp.int32, (_RB, 1), 0)
        m = row < hi
        x = jnp.where(m, xs_ref[...], jnp.bfloat16(0)).astype(jnp.float32)
        out_ref[...] = jnp.dot(x, we_ref[0], preferred_element_type=jnp.float32)


def _run_gemm(meta, xs, w_expert):
    rb = meta[0, :]
    et = meta[1, :]
    lo = meta[2, :]
    hi = meta[3, :]
    grid_spec = pltpu.PrefetchScalarGridSpec(
        num_scalar_prefetch=4,
        grid=(_NT,),
        in_specs=[
            pl.BlockSpec((_RB, _H), lambda t, rb, et, lo, hi: (rb[t], 0)),
            pl.BlockSpec((1, _H, _H), lambda t, rb, et, lo, hi: (et[t], 0, 0)),
        ],
        out_specs=pl.BlockSpec((_RB, _H), lambda t, rb, et, lo, hi: (rb[t], 0)),
    )
    return pl.pallas_call(
        _gemm_body,
        grid_spec=grid_spec,
        out_shape=jax.ShapeDtypeStruct((_PR, _H), jnp.float32),
    )(rb, et, lo, hi, xs, w_expert)


@functools.cache
def _make_sc_combine():
    @functools.partial(
        pl.kernel,
        mesh=_sc_mesh(),
        out_type=jax.ShapeDtypeStruct((_T, _H), jnp.float32),
        scratch_types=[
            pltpu.VMEM((_CW,), jnp.int32),
            pltpu.VMEM((_CW,), jnp.int32),
            pltpu.VMEM((_CW, _H), jnp.float32),
            pltpu.VMEM((_CW, _H), jnp.float32),
            pltpu.SemaphoreType.DMA,
            pltpu.SemaphoreType.DMA,
        ],
    )
    def _sc_combine(ys_hbm, pos_hbm, out_hbm,
                    idx0, idx1, buf0, buf1, sem0, sem1):
        wid = lax.axis_index("s") * 2 + lax.axis_index("c")
        base = wid * _CW
        pltpu.sync_copy(pos_hbm.at[0, pl.ds(base, _CW)], idx0)
        pltpu.sync_copy(pos_hbm.at[1, pl.ds(base, _CW)], idx1)
        c0 = pltpu.async_copy(ys_hbm.at[idx0], buf0, sem0)
        c1 = pltpu.async_copy(ys_hbm.at[idx1], buf1, sem1)
        c0.wait()
        c1.wait()

        def row_body(r, carry):
            for c in range(_H // 16):
                s = pl.ds(c * 16, 16)
                buf0[r, s] = buf0[r, s] + buf1[r, s]
            return carry

        lax.fori_loop(0, _CW, row_body, 0)
        pltpu.sync_copy(buf0, out_hbm.at[pl.ds(base, _CW), :])

    return _sc_combine


def kernel(tokens, W_router, W_expert):
    x = tokens.reshape(_T, _H)
    x0, x1, pos, meta = _run_router(x, W_router)
    x0i = lax.bitcast_convert_type(x0.reshape(_T, _H // 2, 2), jnp.int32)
    x1i = lax.bitcast_convert_type(x1.reshape(_T, _H // 2, 2), jnp.int32)
    xsi = _make_sc_scatter()(x0i, x1i, pos)
    xs = lax.bitcast_convert_type(xsi, jnp.bfloat16).reshape(_PR, _H)
    ys = _run_gemm(meta, xs, W_expert)
    out = _make_sc_combine()(ys, pos)
    return out.reshape(1, _T, _H)


# speedup vs baseline: 3.0771x; 3.0771x over previous
import functools

import jax
import jax.numpy as jnp
from jax import lax
from jax.experimental import pallas as pl
from jax.experimental.pallas import tpu as pltpu
from jax.experimental.pallas import tpu_sc as plsc

_T, _H, _E, _K = 2048, 768, 64, 2
_TK = _T * _K
_RB = 128
_NT = _TK // _RB + _E
_PR = _NT * _RB
_NTP = 128
_NW = 32
_CW = _T // _NW


_HP = _H // 2


def _pack_bf16_cols(y):
    a = lax.bitcast_convert_type(y, jnp.int32)
    r = a + 0x7FFF + (lax.shift_right_logical(a, 16) & 1)
    b = lax.shift_right_logical(r, 16)
    return b[:, :_HP] | lax.shift_left(b[:, _HP:], 16)


def _unpack_bf16_cols(p):
    lo = lax.bitcast_convert_type(lax.shift_left(p, 16), jnp.float32)
    hi = lax.bitcast_convert_type(p & jnp.int32(-65536), jnp.float32)
    return jnp.concatenate([lo, hi], axis=1)


def _router_body(x_ref, wr_ref, x0_ref, x1_ref, pos_ref, meta_ref):
    x = x_ref[...]
    logits = jnp.dot(x, wr_ref[...], preferred_element_type=jnp.float32)
    lane = lax.broadcasted_iota(jnp.int32, (_T, _E), 1)
    m0 = jnp.max(logits, axis=1, keepdims=True)
    i0 = jnp.min(jnp.where(logits == m0, lane, _E), axis=1, keepdims=True)
    oh0 = lane == i0
    masked = jnp.where(oh0, -jnp.inf, logits)
    m1 = jnp.max(masked, axis=1, keepdims=True)
    i1 = jnp.min(jnp.where(masked == m1, lane, _E), axis=1, keepdims=True)
    oh1 = lane == i1
    w0 = jax.nn.sigmoid(m0 - m1)
    w1 = 1.0 - w0
    x0_ref[...] = _pack_bf16_cols(x * w0)
    x1_ref[...] = _pack_bf16_cols(x * w1)

    a = oh0.astype(jnp.float32) + oh1.astype(jnp.float32)
    rows = lax.broadcasted_iota(jnp.int32, (_T, _T), 0)
    cols = lax.broadcasted_iota(jnp.int32, (_T, _T), 1)
    lstrict = (cols < rows).astype(jnp.float32)
    cum = jnp.dot(lstrict, a, preferred_element_type=jnp.float32)
    counts = jnp.sum(a, axis=0)
    counts_i = counts.astype(jnp.int32)
    nb = (counts_i + (_RB - 1)) // _RB
    ef = lax.broadcasted_iota(jnp.int32, (_E, _E), 0)
    ee = lax.broadcasted_iota(jnp.int32, (_E, _E), 1)
    mstrict = (ef < ee).astype(jnp.float32)
    baseblk = jnp.dot(nb.astype(jnp.float32)[None, :], mstrict,
                      preferred_element_type=jnp.float32)[0]
    poffs = baseblk * float(_RB)
    val = poffs[None, :] + cum
    pos0 = jnp.sum(jnp.where(oh0, val, 0.0), axis=1)
    pos1 = jnp.sum(jnp.where(oh1, val, 0.0), axis=1)
    pos_ref[...] = jnp.concatenate(
        [pos0[None, :], pos1[None, :]], axis=0).astype(jnp.int32)

    baseblk_i = baseblk.astype(jnp.int32)
    total = jnp.sum(nb)
    tau = lax.broadcasted_iota(jnp.int32, (_NTP, _E), 0)
    eidx = lax.broadcasted_iota(jnp.int32, (_NTP, _E), 1)
    cand = (nb[None, :] > 0) & (baseblk_i[None, :] <= tau)
    et = jnp.max(jnp.where(cand, eidx, 0), axis=1)
    sel = (eidx == et[:, None]).astype(jnp.int32)
    pe = jnp.sum(sel * (baseblk_i * _RB)[None, :], axis=1)
    ce = jnp.sum(sel * counts_i[None, :], axis=1)
    tvec = lax.broadcasted_iota(jnp.int32, (_NTP,), 0)
    valid = tvec < total
    rb = jnp.where(valid, tvec, total - 1)
    lo = jnp.where(valid, tvec * _RB, 0)
    hi = jnp.where(valid, jnp.minimum(pe + ce, (tvec + 1) * _RB), 0)
    zero = jnp.zeros((4, _NTP), jnp.int32)
    meta_ref[...] = jnp.concatenate(
        [rb[None, :], et[None, :], lo[None, :], hi[None, :], zero], axis=0)


def _run_router(x, w_router):
    return pl.pallas_call(
        _router_body,
        out_shape=(
            jax.ShapeDtypeStruct((_T, _HP), jnp.int32),
            jax.ShapeDtypeStruct((_T, _HP), jnp.int32),
            jax.ShapeDtypeStruct((_K, _T), jnp.int32),
            jax.ShapeDtypeStruct((8, _NTP), jnp.int32),
        ),
    )(x, w_router)


@functools.cache
def _sc_mesh():
    return plsc.VectorSubcoreMesh(core_axis_name="c", subcore_axis_name="s")


@functools.cache
def _make_sc_scatter():
    @functools.partial(
        pl.kernel,
        mesh=_sc_mesh(),
        out_type=jax.ShapeDtypeStruct((_PR, _HP), jnp.int32),
        scratch_types=[
            pltpu.VMEM((_CW,), jnp.int32),
            pltpu.VMEM((_CW,), jnp.int32),
            pltpu.VMEM((_CW, _HP), jnp.int32),
            pltpu.VMEM((_CW, _HP), jnp.int32),
            pltpu.SemaphoreType.DMA,
            pltpu.SemaphoreType.DMA,
        ],
    )
    def _sc_scatter(x0_hbm, x1_hbm, pos_hbm, xs_hbm,
                    idx0, idx1, buf0, buf1, sem0, sem1):
        wid = lax.axis_index("s") * 2 + lax.axis_index("c")
        base = wid * _CW
        pltpu.sync_copy(pos_hbm.at[0, pl.ds(base, _CW)], idx0)
        pltpu.sync_copy(pos_hbm.at[1, pl.ds(base, _CW)], idx1)
        pltpu.sync_copy(x0_hbm.at[pl.ds(base, _CW), :], buf0)
        pltpu.sync_copy(x1_hbm.at[pl.ds(base, _CW), :], buf1)
        c0 = pltpu.async_copy(buf0, xs_hbm.at[idx0], sem0)
        c1 = pltpu.async_copy(buf1, xs_hbm.at[idx1], sem1)
        c0.wait()
        c1.wait()

    return _sc_scatter


def _gemm_body(rb_ref, et_ref, lo_ref, hi_ref, xs_ref, we_ref, out_ref):
    t = pl.program_id(0)
    rb = rb_ref[t]
    lo = lo_ref[t]
    hi = hi_ref[t]

    @pl.when(lo < hi)
    def _compute():
        row = rb * _RB + lax.broadcasted_iota(jnp.int32, (_RB, 1), 0)
        m = row < hi
        x = jnp.where(m, _unpack_bf16_cols(xs_ref[...]), 0.0)
        out_ref[...] = jnp.dot(x, we_ref[0], preferred_element_type=jnp.float32)


def _run_gemm(meta, xs, w_expert):
    rb = meta[0, :]
    et = meta[1, :]
    lo = meta[2, :]
    hi = meta[3, :]
    grid_spec = pltpu.PrefetchScalarGridSpec(
        num_scalar_prefetch=4,
        grid=(_NT,),
        in_specs=[
            pl.BlockSpec((_RB, _HP), lambda t, rb, et, lo, hi: (rb[t], 0)),
            pl.BlockSpec((1, _H, _H), lambda t, rb, et, lo, hi: (et[t], 0, 0)),
        ],
        out_specs=pl.BlockSpec((_RB, _H), lambda t, rb, et, lo, hi: (rb[t], 0)),
    )
    return pl.pallas_call(
        _gemm_body,
        grid_spec=grid_spec,
        out_shape=jax.ShapeDtypeStruct((_PR, _H), jnp.float32),
    )(rb, et, lo, hi, xs, w_expert)


@functools.cache
def _make_sc_combine():
    @functools.partial(
        pl.kernel,
        mesh=_sc_mesh(),
        out_type=jax.ShapeDtypeStruct((_T, _H), jnp.float32),
        scratch_types=[
            pltpu.VMEM((_CW,), jnp.int32),
            pltpu.VMEM((_CW,), jnp.int32),
            pltpu.VMEM((_CW, _H), jnp.float32),
            pltpu.VMEM((_CW, _H), jnp.float32),
            pltpu.SemaphoreType.DMA,
            pltpu.SemaphoreType.DMA,
        ],
    )
    def _sc_combine(ys_hbm, pos_hbm, out_hbm,
                    idx0, idx1, buf0, buf1, sem0, sem1):
        wid = lax.axis_index("s") * 2 + lax.axis_index("c")
        base = wid * _CW
        pltpu.sync_copy(pos_hbm.at[0, pl.ds(base, _CW)], idx0)
        pltpu.sync_copy(pos_hbm.at[1, pl.ds(base, _CW)], idx1)
        c0 = pltpu.async_copy(ys_hbm.at[idx0], buf0, sem0)
        c1 = pltpu.async_copy(ys_hbm.at[idx1], buf1, sem1)
        c0.wait()
        c1.wait()

        def row_body(r, carry):
            for c in range(_H // 16):
                s = pl.ds(c * 16, 16)
                buf0[r, s] = buf0[r, s] + buf1[r, s]
            return carry

        lax.fori_loop(0, _CW, row_body, 0)
        pltpu.sync_copy(buf0, out_hbm.at[pl.ds(base, _CW), :])

    return _sc_combine


def kernel(tokens, W_router, W_expert):
    x = tokens.reshape(_T, _H)
    x0, x1, pos, meta = _run_router(x, W_router)
    xs = _make_sc_scatter()(x0, x1, pos)
    ys = _run_gemm(meta, xs, W_expert)
    out = _make_sc_combine()(ys, pos)
    return out.reshape(1, _T, _H)


# speedup vs baseline: 3.1255x; 1.0157x over previous
import functools

import jax
import jax.numpy as jnp
from jax import lax
from jax.experimental import pallas as pl
from jax.experimental.pallas import tpu as pltpu
from jax.experimental.pallas import tpu_sc as plsc

_T, _H, _E, _K = 2048, 768, 64, 2
_TK = _T * _K
_RB = 128
_NT = _TK // _RB + _E
_PR = _NT * _RB
_NTP = 128
_NW = 32
_CW = _T // _NW


_HP = _H // 2


def _pack_bf16_cols(y):
    a = lax.bitcast_convert_type(y, jnp.int32)
    r = a + 0x7FFF + (lax.shift_right_logical(a, 16) & 1)
    b = lax.shift_right_logical(r, 16)
    return b[:, :_HP] | lax.shift_left(b[:, _HP:], 16)


def _unpack_bf16_cols(p):
    lo = lax.bitcast_convert_type(lax.shift_left(p, 16), jnp.float32)
    hi = lax.bitcast_convert_type(p & jnp.int32(-65536), jnp.float32)
    return jnp.concatenate([lo, hi], axis=1)


def _router_body(x_ref, wr_ref, x0_ref, x1_ref, pos_ref, meta_ref):
    x = x_ref[...]
    logits = jnp.dot(x, wr_ref[...], preferred_element_type=jnp.float32)
    lane = lax.broadcasted_iota(jnp.int32, (_T, _E), 1)
    m0 = jnp.max(logits, axis=1, keepdims=True)
    i0 = jnp.min(jnp.where(logits == m0, lane, _E), axis=1, keepdims=True)
    oh0 = lane == i0
    masked = jnp.where(oh0, -jnp.inf, logits)
    m1 = jnp.max(masked, axis=1, keepdims=True)
    i1 = jnp.min(jnp.where(masked == m1, lane, _E), axis=1, keepdims=True)
    oh1 = lane == i1
    w0 = jax.nn.sigmoid(m0 - m1)
    w1 = 1.0 - w0
    x0_ref[...] = _pack_bf16_cols(x * w0)
    x1_ref[...] = _pack_bf16_cols(x * w1)

    a = oh0.astype(jnp.float32) + oh1.astype(jnp.float32)
    rows = lax.broadcasted_iota(jnp.int32, (_T, _T), 0)
    cols = lax.broadcasted_iota(jnp.int32, (_T, _T), 1)
    lstrict = (cols < rows).astype(jnp.float32)
    cum = jnp.dot(lstrict, a, preferred_element_type=jnp.float32)
    counts = jnp.sum(a, axis=0)
    counts_i = counts.astype(jnp.int32)
    nb = (counts_i + (_RB - 1)) // _RB
    ef = lax.broadcasted_iota(jnp.int32, (_E, _E), 0)
    ee = lax.broadcasted_iota(jnp.int32, (_E, _E), 1)
    mstrict = (ef < ee).astype(jnp.float32)
    baseblk = jnp.dot(nb.astype(jnp.float32)[None, :], mstrict,
                      preferred_element_type=jnp.float32)[0]
    poffs = baseblk * float(_RB)
    val = poffs[None, :] + cum
    pos0 = jnp.sum(jnp.where(oh0, val, 0.0), axis=1)
    pos1 = jnp.sum(jnp.where(oh1, val, 0.0), axis=1)
    pos_ref[...] = jnp.concatenate(
        [pos0[None, :], pos1[None, :]], axis=0).astype(jnp.int32)

    baseblk_i = baseblk.astype(jnp.int32)
    total = jnp.sum(nb)
    tau = lax.broadcasted_iota(jnp.int32, (_NTP, _E), 0)
    eidx = lax.broadcasted_iota(jnp.int32, (_NTP, _E), 1)
    cand = (nb[None, :] > 0) & (baseblk_i[None, :] <= tau)
    et = jnp.max(jnp.where(cand, eidx, 0), axis=1)
    sel = (eidx == et[:, None]).astype(jnp.int32)
    pe = jnp.sum(sel * (baseblk_i * _RB)[None, :], axis=1)
    ce = jnp.sum(sel * counts_i[None, :], axis=1)
    tvec = lax.broadcasted_iota(jnp.int32, (_NTP,), 0)
    valid = tvec < total
    rb = jnp.where(valid, tvec, total - 1)
    lo = jnp.where(valid, tvec * _RB, 0)
    hi = jnp.where(valid, jnp.minimum(pe + ce, (tvec + 1) * _RB), 0)
    zero = jnp.zeros((4, _NTP), jnp.int32)
    meta_ref[...] = jnp.concatenate(
        [rb[None, :], et[None, :], lo[None, :], hi[None, :], zero], axis=0)


def _run_router(x, w_router):
    return pl.pallas_call(
        _router_body,
        out_shape=(
            jax.ShapeDtypeStruct((_T, _HP), jnp.int32),
            jax.ShapeDtypeStruct((_T, _HP), jnp.int32),
            jax.ShapeDtypeStruct((_K, _T), jnp.int32),
            jax.ShapeDtypeStruct((8, _NTP), jnp.int32),
        ),
    )(x, w_router)


@functools.cache
def _sc_mesh():
    return plsc.VectorSubcoreMesh(core_axis_name="c", subcore_axis_name="s")


@functools.cache
def _make_sc_scatter():
    @functools.partial(
        pl.kernel,
        mesh=_sc_mesh(),
        out_type=jax.ShapeDtypeStruct((_PR, _HP), jnp.int32),
        scratch_types=[
            pltpu.VMEM((_CW,), jnp.int32),
            pltpu.VMEM((_CW,), jnp.int32),
            pltpu.VMEM((_CW, _HP), jnp.int32),
            pltpu.VMEM((_CW, _HP), jnp.int32),
            pltpu.SemaphoreType.DMA,
            pltpu.SemaphoreType.DMA,
            pltpu.SemaphoreType.DMA,
            pltpu.SemaphoreType.DMA,
        ],
    )
    def _sc_scatter(x0_hbm, x1_hbm, pos_hbm, xs_hbm,
                    idx0, idx1, buf0, buf1, sem0, sem1, sem2, sem3):
        wid = lax.axis_index("s") * 2 + lax.axis_index("c")
        base = wid * _CW
        ci0 = pltpu.async_copy(pos_hbm.at[0, pl.ds(base, _CW)], idx0, sem0)
        ci1 = pltpu.async_copy(pos_hbm.at[1, pl.ds(base, _CW)], idx1, sem1)
        cb0 = pltpu.async_copy(x0_hbm.at[pl.ds(base, _CW), :], buf0, sem2)
        cb1 = pltpu.async_copy(x1_hbm.at[pl.ds(base, _CW), :], buf1, sem3)
        ci0.wait()
        cb0.wait()
        c0 = pltpu.async_copy(buf0, xs_hbm.at[idx0], sem0)
        ci1.wait()
        cb1.wait()
        c1 = pltpu.async_copy(buf1, xs_hbm.at[idx1], sem1)
        c0.wait()
        c1.wait()

    return _sc_scatter


def _gemm_body(rb_ref, et_ref, lo_ref, hi_ref, xs_ref, we_ref, out_ref):
    t = pl.program_id(0)
    rb = rb_ref[t]
    lo = lo_ref[t]
    hi = hi_ref[t]

    @pl.when(lo < hi)
    def _compute():
        row = rb * _RB + lax.broadcasted_iota(jnp.int32, (_RB, 1), 0)
        m = row < hi
        x = jnp.where(m, _unpack_bf16_cols(xs_ref[...]), 0.0)
        out_ref[...] = jnp.dot(x, we_ref[0], preferred_element_type=jnp.float32)


def _run_gemm(meta, xs, w_expert):
    rb = meta[0, :]
    et = meta[1, :]
    lo = meta[2, :]
    hi = meta[3, :]
    grid_spec = pltpu.PrefetchScalarGridSpec(
        num_scalar_prefetch=4,
        grid=(_NT,),
        in_specs=[
            pl.BlockSpec((_RB, _HP), lambda t, rb, et, lo, hi: (rb[t], 0)),
            pl.BlockSpec((1, _H, _H), lambda t, rb, et, lo, hi: (et[t], 0, 0)),
        ],
        out_specs=pl.BlockSpec((_RB, _H), lambda t, rb, et, lo, hi: (rb[t], 0)),
    )
    return pl.pallas_call(
        _gemm_body,
        grid_spec=grid_spec,
        out_shape=jax.ShapeDtypeStruct((_PR, _H), jnp.float32),
    )(rb, et, lo, hi, xs, w_expert)


@functools.cache
def _make_sc_combine():
    @functools.partial(
        pl.kernel,
        mesh=_sc_mesh(),
        out_type=jax.ShapeDtypeStruct((_T, _H), jnp.float32),
        scratch_types=[
            pltpu.VMEM((_CW,), jnp.int32),
            pltpu.VMEM((_CW,), jnp.int32),
            pltpu.VMEM((_CW, _H), jnp.float32),
            pltpu.VMEM((_CW, _H), jnp.float32),
            pltpu.SemaphoreType.DMA,
            pltpu.SemaphoreType.DMA,
        ],
    )
    def _sc_combine(ys_hbm, pos_hbm, out_hbm,
                    idx0, idx1, buf0, buf1, sem0, sem1):
        wid = lax.axis_index("s") * 2 + lax.axis_index("c")
        base = wid * _CW
        ci0 = pltpu.async_copy(pos_hbm.at[0, pl.ds(base, _CW)], idx0, sem0)
        ci1 = pltpu.async_copy(pos_hbm.at[1, pl.ds(base, _CW)], idx1, sem1)
        ci0.wait()
        c0 = pltpu.async_copy(ys_hbm.at[idx0], buf0, sem0)
        ci1.wait()
        c1 = pltpu.async_copy(ys_hbm.at[idx1], buf1, sem1)
        c0.wait()
        c1.wait()

        def row_body(r, carry):
            for c in range(_H // 16):
                s = pl.ds(c * 16, 16)
                buf0[r, s] = buf0[r, s] + buf1[r, s]
            return carry

        lax.fori_loop(0, _CW, row_body, 0)
        pltpu.sync_copy(buf0, out_hbm.at[pl.ds(base, _CW), :])

    return _sc_combine


def kernel(tokens, W_router, W_expert):
    x = tokens.reshape(_T, _H)
    x0, x1, pos, meta = _run_router(x, W_router)
    xs = _make_sc_scatter()(x0, x1, pos)
    ys = _run_gemm(meta, xs, W_expert)
    out = _make_sc_combine()(ys, pos)
    return out.reshape(1, _T, _H)


# speedup vs baseline: 3.1449x; 1.0062x over previous
import functools

import jax
import jax.numpy as jnp
from jax import lax
from jax.experimental import pallas as pl
from jax.experimental.pallas import tpu as pltpu
from jax.experimental.pallas import tpu_sc as plsc

_T, _H, _E, _K = 2048, 768, 64, 2
_TK = _T * _K
_RB = 128
_NT = _TK // _RB + _E
_PR = _NT * _RB
_NTP = 128
_NW = 32
_CW = _T // _NW


_HP = _H // 2


def _pack_bf16_cols(y):
    a = lax.bitcast_convert_type(y, jnp.int32)
    r = a + 0x7FFF + (lax.shift_right_logical(a, 16) & 1)
    b = lax.shift_right_logical(r, 16)
    return b[:, :_HP] | lax.shift_left(b[:, _HP:], 16)


def _unpack_bf16_cols(p):
    lo = lax.bitcast_convert_type(lax.shift_left(p, 16), jnp.float32)
    hi = lax.bitcast_convert_type(p & jnp.int32(-65536), jnp.float32)
    return jnp.concatenate([lo, hi], axis=1)


def _router_body(x_ref, wr_ref, x0_ref, x1_ref, pos_ref, meta_ref):
    x = x_ref[...]
    logits = jnp.dot(x, wr_ref[...], preferred_element_type=jnp.float32)
    lane = lax.broadcasted_iota(jnp.int32, (_T, _E), 1)
    m0 = jnp.max(logits, axis=1, keepdims=True)
    i0 = jnp.min(jnp.where(logits == m0, lane, _E), axis=1, keepdims=True)
    oh0 = lane == i0
    masked = jnp.where(oh0, -jnp.inf, logits)
    m1 = jnp.max(masked, axis=1, keepdims=True)
    i1 = jnp.min(jnp.where(masked == m1, lane, _E), axis=1, keepdims=True)
    oh1 = lane == i1
    w0 = jax.nn.sigmoid(m0 - m1)
    w1 = 1.0 - w0
    x0_ref[...] = _pack_bf16_cols(x * w0)
    x1_ref[...] = _pack_bf16_cols(x * w1)

    a = oh0.astype(jnp.float32) + oh1.astype(jnp.float32)
    rows = lax.broadcasted_iota(jnp.int32, (_T, _T), 0)
    cols = lax.broadcasted_iota(jnp.int32, (_T, _T), 1)
    lstrict = (cols < rows).astype(jnp.float32)
    cum = jnp.dot(lstrict, a, preferred_element_type=jnp.float32)
    counts = jnp.sum(a, axis=0)
    counts_i = counts.astype(jnp.int32)
    nb = (counts_i + (_RB - 1)) // _RB
    ef = lax.broadcasted_iota(jnp.int32, (_E, _E), 0)
    ee = lax.broadcasted_iota(jnp.int32, (_E, _E), 1)
    mstrict = (ef < ee).astype(jnp.float32)
    baseblk = jnp.dot(nb.astype(jnp.float32)[None, :], mstrict,
                      preferred_element_type=jnp.float32)[0]
    poffs = baseblk * float(_RB)
    val = poffs[None, :] + cum
    pos0 = jnp.sum(jnp.where(oh0, val, 0.0), axis=1)
    pos1 = jnp.sum(jnp.where(oh1, val, 0.0), axis=1)
    pos_ref[...] = jnp.concatenate(
        [pos0[None, :], pos1[None, :]], axis=0).astype(jnp.int32)

    baseblk_i = baseblk.astype(jnp.int32)
    total = jnp.sum(nb)
    tau = lax.broadcasted_iota(jnp.int32, (_NTP, _E), 0)
    eidx = lax.broadcasted_iota(jnp.int32, (_NTP, _E), 1)
    cand = (nb[None, :] > 0) & (baseblk_i[None, :] <= tau)
    et = jnp.max(jnp.where(cand, eidx, 0), axis=1)
    sel = (eidx == et[:, None]).astype(jnp.int32)
    pe = jnp.sum(sel * (baseblk_i * _RB)[None, :], axis=1)
    ce = jnp.sum(sel * counts_i[None, :], axis=1)
    tvec = lax.broadcasted_iota(jnp.int32, (_NTP,), 0)
    valid = tvec < total
    rb = jnp.where(valid, tvec, total - 1)
    lo = jnp.where(valid, tvec * _RB, 0)
    hi = jnp.where(valid, jnp.minimum(pe + ce, (tvec + 1) * _RB), 0)
    zero = jnp.zeros((4, _NTP), jnp.int32)
    meta_ref[...] = jnp.concatenate(
        [rb[None, :], et[None, :], lo[None, :], hi[None, :], zero], axis=0)


def _run_router(x, w_router):
    return pl.pallas_call(
        _router_body,
        out_shape=(
            jax.ShapeDtypeStruct((_T, _HP), jnp.int32),
            jax.ShapeDtypeStruct((_T, _HP), jnp.int32),
            jax.ShapeDtypeStruct((_K, _T), jnp.int32),
            jax.ShapeDtypeStruct((8, _NTP), jnp.int32),
        ),
    )(x, w_router)


@functools.cache
def _sc_mesh():
    return plsc.VectorSubcoreMesh(core_axis_name="c", subcore_axis_name="s")


@functools.cache
def _make_sc_scatter():
    @functools.partial(
        pl.kernel,
        mesh=_sc_mesh(),
        out_type=jax.ShapeDtypeStruct((_PR, _HP), jnp.int32),
        scratch_types=[
            pltpu.VMEM((_CW,), jnp.int32),
            pltpu.VMEM((_CW,), jnp.int32),
            pltpu.VMEM((_CW, _HP), jnp.int32),
            pltpu.VMEM((_CW, _HP), jnp.int32),
            pltpu.SemaphoreType.DMA,
            pltpu.SemaphoreType.DMA,
            pltpu.SemaphoreType.DMA,
            pltpu.SemaphoreType.DMA,
        ],
    )
    def _sc_scatter(x0_hbm, x1_hbm, pos_hbm, xs_hbm,
                    idx0, idx1, buf0, buf1, sem0, sem1, sem2, sem3):
        wid = lax.axis_index("s") * 2 + lax.axis_index("c")
        base = wid * _CW
        ci0 = pltpu.async_copy(pos_hbm.at[0, pl.ds(base, _CW)], idx0, sem0)
        ci1 = pltpu.async_copy(pos_hbm.at[1, pl.ds(base, _CW)], idx1, sem1)
        cb0 = pltpu.async_copy(x0_hbm.at[pl.ds(base, _CW), :], buf0, sem2)
        cb1 = pltpu.async_copy(x1_hbm.at[pl.ds(base, _CW), :], buf1, sem3)
        ci0.wait()
        cb0.wait()
        c0 = pltpu.async_copy(buf0, xs_hbm.at[idx0], sem0)
        ci1.wait()
        cb1.wait()
        c1 = pltpu.async_copy(buf1, xs_hbm.at[idx1], sem1)
        c0.wait()
        c1.wait()

    return _sc_scatter


def _gemm_body(rb_ref, et_ref, lo_ref, hi_ref, xs_ref, we_ref, out_ref):
    t = pl.program_id(0)
    rb = rb_ref[t]
    lo = lo_ref[t]
    hi = hi_ref[t]

    @pl.when(lo < hi)
    def _compute():
        row = rb * _RB + lax.broadcasted_iota(jnp.int32, (_RB, 1), 0)
        m = row < hi
        x = jnp.where(m, _unpack_bf16_cols(xs_ref[...]), 0.0)
        out_ref[...] = jnp.dot(x, we_ref[0], preferred_element_type=jnp.float32)


def _run_gemm(meta, xs, w_expert):
    rb = meta[0, :]
    et = meta[1, :]
    lo = meta[2, :]
    hi = meta[3, :]
    grid_spec = pltpu.PrefetchScalarGridSpec(
        num_scalar_prefetch=4,
        grid=(_NT,),
        in_specs=[
            pl.BlockSpec((_RB, _HP), lambda t, rb, et, lo, hi: (rb[t], 0)),
            pl.BlockSpec((1, _H, _H), lambda t, rb, et, lo, hi: (et[t], 0, 0)),
        ],
        out_specs=pl.BlockSpec((_RB, _H), lambda t, rb, et, lo, hi: (rb[t], 0)),
    )
    return pl.pallas_call(
        _gemm_body,
        grid_spec=grid_spec,
        out_shape=jax.ShapeDtypeStruct((_PR, _H), jnp.float32),
    )(rb, et, lo, hi, xs, w_expert)


@functools.cache
def _make_sc_combine():
    hw = _CW // 2

    @functools.partial(
        pl.kernel,
        mesh=_sc_mesh(),
        out_type=jax.ShapeDtypeStruct((_T, _H), jnp.float32),
        scratch_types=[
            pltpu.VMEM((hw,), jnp.int32),
            pltpu.VMEM((hw,), jnp.int32),
            pltpu.VMEM((hw,), jnp.int32),
            pltpu.VMEM((hw,), jnp.int32),
            pltpu.VMEM((_CW, _H), jnp.float32),
            pltpu.VMEM((_CW, _H), jnp.float32),
            pltpu.SemaphoreType.DMA,
            pltpu.SemaphoreType.DMA,
            pltpu.SemaphoreType.DMA,
            pltpu.SemaphoreType.DMA,
            pltpu.SemaphoreType.DMA,
        ],
    )
    def _sc_combine(ys_hbm, pos_hbm, out_hbm,
                    idx0a, idx0b, idx1a, idx1b, buf0, buf1,
                    sem0, sem1, sem2, sem3, semo):
        wid = lax.axis_index("s") * 2 + lax.axis_index("c")
        base = wid * _CW
        ca = pltpu.async_copy(pos_hbm.at[0, pl.ds(base, hw)], idx0a, sem0)
        cb = pltpu.async_copy(pos_hbm.at[1, pl.ds(base, hw)], idx1a, sem1)
        cc = pltpu.async_copy(pos_hbm.at[0, pl.ds(base + hw, hw)], idx0b, sem2)
        cd = pltpu.async_copy(pos_hbm.at[1, pl.ds(base + hw, hw)], idx1b, sem3)
        ca.wait()
        g0a = pltpu.async_copy(ys_hbm.at[idx0a], buf0.at[pl.ds(0, hw), :], sem0)
        cb.wait()
        g1a = pltpu.async_copy(ys_hbm.at[idx1a], buf1.at[pl.ds(0, hw), :], sem1)
        cc.wait()
        g0b = pltpu.async_copy(ys_hbm.at[idx0b], buf0.at[pl.ds(hw, hw), :], sem2)
        cd.wait()
        g1b = pltpu.async_copy(ys_hbm.at[idx1b], buf1.at[pl.ds(hw, hw), :], sem3)

        def row_body(r, carry):
            for c in range(_H // 16):
                s = pl.ds(c * 16, 16)
                buf0[r, s] = buf0[r, s] + buf1[r, s]
            return carry

        g0a.wait()
        g1a.wait()
        lax.fori_loop(0, hw, row_body, 0)
        co = pltpu.async_copy(buf0.at[pl.ds(0, hw), :],
                              out_hbm.at[pl.ds(base, hw), :], semo)
        g0b.wait()
        g1b.wait()
        lax.fori_loop(hw, _CW, row_body, 0)
        co.wait()
        pltpu.sync_copy(buf0.at[pl.ds(hw, hw), :],
                        out_hbm.at[pl.ds(base + hw, hw), :])

    return _sc_combine


def kernel(tokens, W_router, W_expert):
    x = tokens.reshape(_T, _H)
    x0, x1, pos, meta = _run_router(x, W_router)
    xs = _make_sc_scatter()(x0, x1, pos)
    ys = _run_gemm(meta, xs, W_expert)
    out = _make_sc_combine()(ys, pos)
    return out.reshape(1, _T, _H)
